# baseline (speedup 1.0000x reference)
import functools

import jax
import jax.numpy as jnp
from jax import lax
from jax.experimental import pallas as pl
from jax.experimental.pallas import tpu as pltpu
from jax.experimental.pallas import tpu_sc as plsc

CUTOFF = 5.0
EPSILON = 0.1
SIGMA = 1.0
N_ATOMS = 100000
N_EDGES = 6400000

NC = 2
NS = 16
NW = NC * NS
LANES = 16

CHUNK = 2048
TOTAL_CHUNKS = N_EDGES // CHUNK
MAX_CHUNKS_PER_TILE = -(-TOTAL_CHUNKS // NW)
PAIRS = (MAX_CHUNKS_PER_TILE + 1) // 2
GROUPS = CHUNK // LANES

NA_PAD = 100352
ROUNDS = 28
BLOCK = NA_PAD // ROUNDS
MSLICE = BLOCK // NS

_SHIFT = 4.0 * EPSILON * ((SIGMA / CUTOFF) ** 12 - (SIGMA / CUTOFF) ** 6)
HALF_SHIFT = 0.5 * _SHIFT
TWO_EPS = 2.0 * EPSILON


def _lj_body(dist_hbm, i_hbm, j_hbm, out_hbm,
             dbuf0, dbuf1, ibuf0, ibuf1, jbuf0, jbuf1, acc, tbuf, abuf,
             shared, sd0, sd1, si0, si1, sj0, sj1):
    c = lax.axis_index("c")
    s = lax.axis_index("s")
    wid = s * NC + c

    dbufs, ibufs = [dbuf0, dbuf1], [ibuf0, ibuf1]
    jbufs = [jbuf0, jbuf1]
    sds, sis, sjs = [sd0, sd1], [si0, si1], [sj0, sj1]

    zero16 = jnp.zeros((LANES,), jnp.float32)

    @plsc.parallel_loop(0, NA_PAD, step=LANES, unroll=8)
    def zero_body(k0):
        acc[pl.ds(k0, LANES)] = zero16

    def cid_of(k):
        return k * NW + wid

    def issue_in(k, p):
        @pl.when(cid_of(k) < TOTAL_CHUNKS)
        def _():
            base = cid_of(k) * CHUNK
            pltpu.async_copy(dist_hbm.at[:, pl.ds(base, CHUNK)], dbufs[p], sds[p])
            pltpu.async_copy(i_hbm.at[pl.ds(base, CHUNK)], ibufs[p], sis[p])
            pltpu.async_copy(j_hbm.at[pl.ds(base, CHUNK)], jbufs[p], sjs[p])

    def wait_in(k, p):
        @pl.when(cid_of(k) < TOTAL_CHUNKS)
        def _():
            base = cid_of(k) * CHUNK
            pltpu.make_async_copy(dist_hbm.at[:, pl.ds(base, CHUNK)], dbufs[p], sds[p]).wait()
            pltpu.make_async_copy(i_hbm.at[pl.ds(base, CHUNK)], ibufs[p], sis[p]).wait()
            pltpu.make_async_copy(j_hbm.at[pl.ds(base, CHUNK)], jbufs[p], sjs[p]).wait()

    def step(k, p):
        issue_in(k + 1, 1 - p)
        wait_in(k, p)

        @pl.when(cid_of(k) < TOTAL_CHUNKS)
        def _():
            dbuf, ibuf, jbuf = dbufs[p], ibufs[p], jbufs[p]

            @plsc.parallel_loop(0, CHUNK, step=LANES, unroll=4)
            def vec_body(v0):
                sl = pl.ds(v0, LANES)
                dx = dbuf[0, sl]
                dy = dbuf[1, sl]
                dz = dbuf[2, sl]
                r2 = dx * dx + dy * dy + dz * dz
                inv = 1.0 / r2
                s6 = inv * inv * inv
                he = TWO_EPS * (s6 * s6 - s6) - HALF_SHIFT
                plsc.addupdate_scatter(acc, [ibuf[sl]], he)
                plsc.addupdate_scatter(acc, [jbuf[sl]], he)

    issue_in(0, 0)

    def pair_body(m, carry):
        step(2 * m, 0)
        step(2 * m + 1, 1)
        return carry

    lax.fori_loop(0, PAIRS, pair_body, 0)

    def merge_round(r, carry):
        pltpu.sync_copy(acc.at[pl.ds(r * BLOCK, BLOCK)],
                        shared.at[pl.ds(s * BLOCK, BLOCK)])
        plsc.subcore_barrier()

        moff = s * MSLICE
        pltpu.sync_copy(shared.at[pl.ds(moff, MSLICE)], abuf)

        def merge_tile(t, carry2):
            pltpu.sync_copy(shared.at[pl.ds(t * BLOCK + moff, MSLICE)], tbuf)

            def add_body(k, carry3):
                sl = pl.ds(k * LANES, LANES)
                abuf[sl] += tbuf[sl]
                return carry3

            lax.fori_loop(0, MSLICE // LANES, add_body, 0, unroll=8)
            return carry2

        lax.fori_loop(1, NS, merge_tile, 0)

        pltpu.sync_copy(
            abuf, out_hbm.at[pl.ds(c * NA_PAD + r * BLOCK + moff, MSLICE)])
        plsc.subcore_barrier()
        return carry

    lax.fori_loop(0, ROUNDS, merge_round, 0)


@functools.partial(
    pl.kernel,
    out_type=jax.ShapeDtypeStruct((NC * NA_PAD,), jnp.float32),
    mesh=plsc.VectorSubcoreMesh(core_axis_name="c", subcore_axis_name="s"),
    compiler_params=pltpu.CompilerParams(needs_layout_passes=False),
    scratch_types=[
        pltpu.VMEM((3, CHUNK), jnp.float32),
        pltpu.VMEM((3, CHUNK), jnp.float32),
        pltpu.VMEM((CHUNK,), jnp.int32),
        pltpu.VMEM((CHUNK,), jnp.int32),
        pltpu.VMEM((CHUNK,), jnp.int32),
        pltpu.VMEM((CHUNK,), jnp.int32),
        pltpu.VMEM((NA_PAD,), jnp.float32),
        pltpu.VMEM((MSLICE,), jnp.float32),
        pltpu.VMEM((MSLICE,), jnp.float32),
        pltpu.VMEM_SHARED((NS * BLOCK,), jnp.float32),
        pltpu.SemaphoreType.DMA,
        pltpu.SemaphoreType.DMA,
        pltpu.SemaphoreType.DMA,
        pltpu.SemaphoreType.DMA,
        pltpu.SemaphoreType.DMA,
        pltpu.SemaphoreType.DMA,
    ],
)
def _lj_kernel(dist_hbm, i_hbm, j_hbm, out_hbm, *scratch):
    _lj_body(dist_hbm, i_hbm, j_hbm, out_hbm, *scratch)


def kernel(distances, all_i, all_j):
    dist_t = distances.T
    partials = _lj_kernel(dist_t, all_i, all_j)
    partials = partials.reshape(NC, NA_PAD)
    energy = partials[0, :N_ATOMS] + partials[1, :N_ATOMS]
    return energy.reshape(-1, 1)


# speedup vs baseline: 1.2779x; 1.2779x over previous
import functools

import jax
import jax.numpy as jnp
from jax import lax
from jax.experimental import pallas as pl
from jax.experimental.pallas import tpu as pltpu
from jax.experimental.pallas import tpu_sc as plsc

CUTOFF = 5.0
EPSILON = 0.1
SIGMA = 1.0
N_ATOMS = 100000
N_EDGES = 6400000

NC = 2
NS = 16
NW = NC * NS
LANES = 16

CHUNK = 2048
TOTAL_CHUNKS = N_EDGES // CHUNK
MAX_CHUNKS_PER_TILE = -(-TOTAL_CHUNKS // NW)
PAIRS = (MAX_CHUNKS_PER_TILE + 1) // 2
GROUPS = CHUNK // LANES

NA_PAD = 100352
ROUNDS = 49
BLOCK = NA_PAD // ROUNDS
MSLICE = BLOCK // NS

_SHIFT = 4.0 * EPSILON * ((SIGMA / CUTOFF) ** 12 - (SIGMA / CUTOFF) ** 6)
HALF_SHIFT = 0.5 * _SHIFT
TWO_EPS = 2.0 * EPSILON


def _lj_body(dist_hbm, i_hbm, j_hbm, out_hbm,
             dbuf0, dbuf1, ibuf0, ibuf1, jbuf0, jbuf1, acc, rbuf, abuf,
             shared, sd0, sd1, si0, si1, sj0, sj1, sm):
    c = lax.axis_index("c")
    s = lax.axis_index("s")
    wid = s * NC + c

    dbufs, ibufs = [dbuf0, dbuf1], [ibuf0, ibuf1]
    jbufs = [jbuf0, jbuf1]
    sds, sis, sjs = [sd0, sd1], [si0, si1], [sj0, sj1]

    zero16 = jnp.zeros((LANES,), jnp.float32)

    @plsc.parallel_loop(0, NA_PAD, step=LANES, unroll=8)
    def zero_body(k0):
        acc[pl.ds(k0, LANES)] = zero16

    def cid_of(k):
        return k * NW + wid

    def issue_in(k, p):
        @pl.when(cid_of(k) < TOTAL_CHUNKS)
        def _():
            base = cid_of(k) * CHUNK
            pltpu.async_copy(dist_hbm.at[:, pl.ds(base, CHUNK)], dbufs[p], sds[p])
            pltpu.async_copy(i_hbm.at[pl.ds(base, CHUNK)], ibufs[p], sis[p])
            pltpu.async_copy(j_hbm.at[pl.ds(base, CHUNK)], jbufs[p], sjs[p])

    def wait_in(k, p):
        @pl.when(cid_of(k) < TOTAL_CHUNKS)
        def _():
            base = cid_of(k) * CHUNK
            pltpu.make_async_copy(dist_hbm.at[:, pl.ds(base, CHUNK)], dbufs[p], sds[p]).wait()
            pltpu.make_async_copy(i_hbm.at[pl.ds(base, CHUNK)], ibufs[p], sis[p]).wait()
            pltpu.make_async_copy(j_hbm.at[pl.ds(base, CHUNK)], jbufs[p], sjs[p]).wait()

    def step(k, p):
        issue_in(k + 1, 1 - p)
        wait_in(k, p)

        @pl.when(cid_of(k) < TOTAL_CHUNKS)
        def _():
            dbuf, ibuf, jbuf = dbufs[p], ibufs[p], jbufs[p]

            @plsc.parallel_loop(0, CHUNK, step=LANES, unroll=4)
            def vec_body(v0):
                sl = pl.ds(v0, LANES)
                dx = dbuf[0, sl]
                dy = dbuf[1, sl]
                dz = dbuf[2, sl]
                r2 = dx * dx + dy * dy + dz * dz
                inv = 1.0 / r2
                s6 = inv * inv * inv
                he = TWO_EPS * (s6 * s6 - s6) - HALF_SHIFT
                plsc.addupdate_scatter(acc, [ibuf[sl]], he)
                plsc.addupdate_scatter(acc, [jbuf[sl]], he)

    issue_in(0, 0)

    def pair_body(m, carry):
        step(2 * m, 0)
        step(2 * m + 1, 1)
        return carry

    lax.fori_loop(0, PAIRS, pair_body, 0)

    def merge_round(r, carry):
        pltpu.sync_copy(acc.at[pl.ds(r * BLOCK, BLOCK)],
                        shared.at[pl.ds(s * BLOCK, BLOCK)])
        plsc.subcore_barrier()

        moff = s * MSLICE
        for t in range(NS):
            pltpu.async_copy(shared.at[pl.ds(t * BLOCK + moff, MSLICE)],
                             rbuf.at[pl.ds(t * MSLICE, MSLICE)], sm)
        for t in range(NS):
            pltpu.make_async_copy(shared.at[pl.ds(t * BLOCK + moff, MSLICE)],
                                  rbuf.at[pl.ds(t * MSLICE, MSLICE)], sm).wait()

        @plsc.parallel_loop(0, MSLICE, step=LANES, unroll=2)
        def add_body(k0):
            v = rbuf[pl.ds(k0, LANES)]
            for t in range(1, NS):
                v += rbuf[pl.ds(t * MSLICE + k0, LANES)]
            abuf[pl.ds(k0, LANES)] = v

        pltpu.sync_copy(
            abuf, out_hbm.at[pl.ds(c * NA_PAD + r * BLOCK + moff, MSLICE)])
        plsc.subcore_barrier()
        return carry

    lax.fori_loop(0, ROUNDS, merge_round, 0)


@functools.partial(
    pl.kernel,
    out_type=jax.ShapeDtypeStruct((NC * NA_PAD,), jnp.float32),
    mesh=plsc.VectorSubcoreMesh(core_axis_name="c", subcore_axis_name="s"),
    compiler_params=pltpu.CompilerParams(needs_layout_passes=False),
    scratch_types=[
        pltpu.VMEM((3, CHUNK), jnp.float32),
        pltpu.VMEM((3, CHUNK), jnp.float32),
        pltpu.VMEM((CHUNK,), jnp.int32),
        pltpu.VMEM((CHUNK,), jnp.int32),
        pltpu.VMEM((CHUNK,), jnp.int32),
        pltpu.VMEM((CHUNK,), jnp.int32),
        pltpu.VMEM((NA_PAD,), jnp.float32),
        pltpu.VMEM((NS * MSLICE,), jnp.float32),
        pltpu.VMEM((MSLICE,), jnp.float32),
        pltpu.VMEM_SHARED((NS * BLOCK,), jnp.float32),
        pltpu.SemaphoreType.DMA,
        pltpu.SemaphoreType.DMA,
        pltpu.SemaphoreType.DMA,
        pltpu.SemaphoreType.DMA,
        pltpu.SemaphoreType.DMA,
        pltpu.SemaphoreType.DMA,
        pltpu.SemaphoreType.DMA,
    ],
)
def _lj_kernel(dist_hbm, i_hbm, j_hbm, out_hbm, *scratch):
    _lj_body(dist_hbm, i_hbm, j_hbm, out_hbm, *scratch)


def kernel(distances, all_i, all_j):
    dist_t = distances.T
    partials = _lj_kernel(dist_t, all_i, all_j)
    partials = partials.reshape(NC, NA_PAD)
    energy = partials[0, :N_ATOMS] + partials[1, :N_ATOMS]
    return energy.reshape(-1, 1)


# speedup vs baseline: 1.2906x; 1.0099x over previous
import functools

import jax
import jax.numpy as jnp
from jax import lax
from jax.experimental import pallas as pl
from jax.experimental.pallas import tpu as pltpu
from jax.experimental.pallas import tpu_sc as plsc

CUTOFF = 5.0
EPSILON = 0.1
SIGMA = 1.0
N_ATOMS = 100000
N_EDGES = 6400000

NC = 2
NS = 16
NW = NC * NS
LANES = 16

CHUNK = 2048
TOTAL_CHUNKS = N_EDGES // CHUNK
MAX_CHUNKS_PER_TILE = -(-TOTAL_CHUNKS // NW)
PAIRS = (MAX_CHUNKS_PER_TILE + 1) // 2
GROUPS = CHUNK // LANES

NA_PAD = 100352
ROUNDS = 49
BLOCK = NA_PAD // ROUNDS
MSLICE = BLOCK // NS

_SHIFT = 4.0 * EPSILON * ((SIGMA / CUTOFF) ** 12 - (SIGMA / CUTOFF) ** 6)
HALF_SHIFT = 0.5 * _SHIFT
TWO_EPS = 2.0 * EPSILON


def _lj_body(dist_hbm, i_hbm, j_hbm, out_hbm,
             dbuf0, dbuf1, ibuf0, ibuf1, jbuf0, jbuf1, acc, rbuf, abuf,
             shared, sd0, sd1, si0, si1, sj0, sj1, sm):
    c = lax.axis_index("c")
    s = lax.axis_index("s")
    wid = s * NC + c

    dbufs, ibufs = [dbuf0, dbuf1], [ibuf0, ibuf1]
    jbufs = [jbuf0, jbuf1]
    sds, sis, sjs = [sd0, sd1], [si0, si1], [sj0, sj1]

    def cid_of(k):
        return k * NW + wid

    def issue_in(k, p):
        @pl.when(cid_of(k) < TOTAL_CHUNKS)
        def _():
            base = cid_of(k) * CHUNK
            pltpu.async_copy(dist_hbm.at[:, pl.ds(base, CHUNK)], dbufs[p], sds[p])
            pltpu.async_copy(i_hbm.at[pl.ds(base, CHUNK)], ibufs[p], sis[p])
            pltpu.async_copy(j_hbm.at[pl.ds(base, CHUNK)], jbufs[p], sjs[p])

    def wait_in(k, p):
        @pl.when(cid_of(k) < TOTAL_CHUNKS)
        def _():
            base = cid_of(k) * CHUNK
            pltpu.make_async_copy(dist_hbm.at[:, pl.ds(base, CHUNK)], dbufs[p], sds[p]).wait()
            pltpu.make_async_copy(i_hbm.at[pl.ds(base, CHUNK)], ibufs[p], sis[p]).wait()
            pltpu.make_async_copy(j_hbm.at[pl.ds(base, CHUNK)], jbufs[p], sjs[p]).wait()

    def step(k, p):
        issue_in(k + 1, 1 - p)
        wait_in(k, p)

        @pl.when(cid_of(k) < TOTAL_CHUNKS)
        def _():
            dbuf, ibuf, jbuf = dbufs[p], ibufs[p], jbufs[p]

            @plsc.parallel_loop(0, CHUNK, step=LANES, unroll=4)
            def vec_body(v0):
                sl = pl.ds(v0, LANES)
                dx = dbuf[0, sl]
                dy = dbuf[1, sl]
                dz = dbuf[2, sl]
                r2 = dx * dx + dy * dy + dz * dz
                inv = 1.0 / r2
                s6 = inv * inv * inv
                he = TWO_EPS * (s6 * s6 - s6) - HALF_SHIFT
                plsc.addupdate_scatter(acc, [ibuf[sl]], he)
                plsc.addupdate_scatter(acc, [jbuf[sl]], he)

    issue_in(0, 0)

    zero16 = jnp.zeros((LANES,), jnp.float32)

    @plsc.parallel_loop(0, NA_PAD, step=LANES, unroll=8)
    def zero_body(k0):
        acc[pl.ds(k0, LANES)] = zero16

    def pair_body(m, carry):
        step(2 * m, 0)
        step(2 * m + 1, 1)
        return carry

    lax.fori_loop(0, PAIRS, pair_body, 0)

    def merge_round(r, carry):
        pltpu.sync_copy(acc.at[pl.ds(r * BLOCK, BLOCK)],
                        shared.at[pl.ds(s * BLOCK, BLOCK)])
        plsc.subcore_barrier()

        moff = s * MSLICE
        for t in range(NS):
            pltpu.async_copy(shared.at[pl.ds(t * BLOCK + moff, MSLICE)],
                             rbuf.at[pl.ds(t * MSLICE, MSLICE)], sm)
        for t in range(NS):
            pltpu.make_async_copy(shared.at[pl.ds(t * BLOCK + moff, MSLICE)],
                                  rbuf.at[pl.ds(t * MSLICE, MSLICE)], sm).wait()

        @plsc.parallel_loop(0, MSLICE, step=LANES, unroll=2)
        def add_body(k0):
            v = rbuf[pl.ds(k0, LANES)]
            for t in range(1, NS):
                v += rbuf[pl.ds(t * MSLICE + k0, LANES)]
            abuf[pl.ds(k0, LANES)] = v

        pltpu.sync_copy(
            abuf, out_hbm.at[pl.ds(c * NA_PAD + r * BLOCK + moff, MSLICE)])
        plsc.subcore_barrier()
        return carry

    lax.fori_loop(0, ROUNDS, merge_round, 0)


@functools.partial(
    pl.kernel,
    out_type=jax.ShapeDtypeStruct((NC * NA_PAD,), jnp.float32),
    mesh=plsc.VectorSubcoreMesh(core_axis_name="c", subcore_axis_name="s"),
    compiler_params=pltpu.CompilerParams(needs_layout_passes=False),
    scratch_types=[
        pltpu.VMEM((3, CHUNK), jnp.float32),
        pltpu.VMEM((3, CHUNK), jnp.float32),
        pltpu.VMEM((CHUNK,), jnp.int32),
        pltpu.VMEM((CHUNK,), jnp.int32),
        pltpu.VMEM((CHUNK,), jnp.int32),
        pltpu.VMEM((CHUNK,), jnp.int32),
        pltpu.VMEM((NA_PAD,), jnp.float32),
        pltpu.VMEM((NS * MSLICE,), jnp.float32),
        pltpu.VMEM((MSLICE,), jnp.float32),
        pltpu.VMEM_SHARED((NS * BLOCK,), jnp.float32),
        pltpu.SemaphoreType.DMA,
        pltpu.SemaphoreType.DMA,
        pltpu.SemaphoreType.DMA,
        pltpu.SemaphoreType.DMA,
        pltpu.SemaphoreType.DMA,
        pltpu.SemaphoreType.DMA,
        pltpu.SemaphoreType.DMA,
    ],
)
def _lj_kernel(dist_hbm, i_hbm, j_hbm, out_hbm, *scratch):
    _lj_body(dist_hbm, i_hbm, j_hbm, out_hbm, *scratch)


def kernel(distances, all_i, all_j):
    dist_t = distances.T
    partials = _lj_kernel(dist_t, all_i, all_j)
    partials = partials.reshape(NC, NA_PAD)
    energy = partials[0, :N_ATOMS] + partials[1, :N_ATOMS]
    return energy.reshape(-1, 1)


# speedup vs baseline: 1.3023x; 1.0091x over previous
import functools

import jax
import jax.numpy as jnp
from jax import lax
from jax.experimental import pallas as pl
from jax.experimental.pallas import tpu as pltpu
from jax.experimental.pallas import tpu_sc as plsc

CUTOFF = 5.0
EPSILON = 0.1
SIGMA = 1.0
N_ATOMS = 100000
N_EDGES = 6400000

NC = 2
NS = 16
NW = NC * NS
LANES = 16

CHUNK = 2048
TOTAL_CHUNKS = N_EDGES // CHUNK
MAX_CHUNKS_PER_TILE = -(-TOTAL_CHUNKS // NW)
PAIRS = (MAX_CHUNKS_PER_TILE + 1) // 2
GROUPS = CHUNK // LANES

NA_PAD = 100352
ROUNDS = 49
BLOCK = NA_PAD // ROUNDS
MSLICE = BLOCK // NS

_SHIFT = 4.0 * EPSILON * ((SIGMA / CUTOFF) ** 12 - (SIGMA / CUTOFF) ** 6)
HALF_SHIFT = 0.5 * _SHIFT
TWO_EPS = 2.0 * EPSILON


def _lj_body(dist_hbm, i_hbm, j_hbm, out_hbm,
             dbuf0, dbuf1, ibuf0, ibuf1, jbuf0, jbuf1, acc, rbuf,
             shared, sd0, sd1, si0, si1, sj0, sj1, sm, sp):
    c = lax.axis_index("c")
    s = lax.axis_index("s")
    wid = s * NC + c

    dbufs, ibufs = [dbuf0, dbuf1], [ibuf0, ibuf1]
    jbufs = [jbuf0, jbuf1]
    sds, sis, sjs = [sd0, sd1], [si0, si1], [sj0, sj1]

    def cid_of(k):
        return k * NW + wid

    def issue_in(k, p):
        @pl.when(cid_of(k) < TOTAL_CHUNKS)
        def _():
            base = cid_of(k) * CHUNK
            pltpu.async_copy(dist_hbm.at[:, pl.ds(base, CHUNK)], dbufs[p], sds[p])
            pltpu.async_copy(i_hbm.at[pl.ds(base, CHUNK)], ibufs[p], sis[p])
            pltpu.async_copy(j_hbm.at[pl.ds(base, CHUNK)], jbufs[p], sjs[p])

    def wait_in(k, p):
        @pl.when(cid_of(k) < TOTAL_CHUNKS)
        def _():
            base = cid_of(k) * CHUNK
            pltpu.make_async_copy(dist_hbm.at[:, pl.ds(base, CHUNK)], dbufs[p], sds[p]).wait()
            pltpu.make_async_copy(i_hbm.at[pl.ds(base, CHUNK)], ibufs[p], sis[p]).wait()
            pltpu.make_async_copy(j_hbm.at[pl.ds(base, CHUNK)], jbufs[p], sjs[p]).wait()

    def step(k, p):
        issue_in(k + 1, 1 - p)
        wait_in(k, p)

        @pl.when(cid_of(k) < TOTAL_CHUNKS)
        def _():
            dbuf, ibuf, jbuf = dbufs[p], ibufs[p], jbufs[p]

            @plsc.parallel_loop(0, CHUNK, step=LANES, unroll=4)
            def vec_body(v0):
                sl = pl.ds(v0, LANES)
                dx = dbuf[0, sl]
                dy = dbuf[1, sl]
                dz = dbuf[2, sl]
                r2 = dx * dx + dy * dy + dz * dz
                inv = 1.0 / r2
                s6 = inv * inv * inv
                he = TWO_EPS * (s6 * s6 - s6) - HALF_SHIFT
                plsc.addupdate_scatter(acc, [ibuf[sl]], he)
                plsc.addupdate_scatter(acc, [jbuf[sl]], he)

    issue_in(0, 0)

    zero16 = jnp.zeros((LANES,), jnp.float32)

    @plsc.parallel_loop(0, NA_PAD, step=LANES, unroll=8)
    def zero_body(k0):
        acc[pl.ds(k0, LANES)] = zero16

    def pair_body(m, carry):
        step(2 * m, 0)
        step(2 * m + 1, 1)
        return carry

    lax.fori_loop(0, PAIRS, pair_body, 0)

    def pub(r, pb):
        @pl.when(r < ROUNDS)
        def _():
            pltpu.async_copy(acc.at[pl.ds(r * BLOCK, BLOCK)],
                             shared.at[pl.ds(pb * (NS * BLOCK) + s * BLOCK, BLOCK)],
                             sp)

    def pub_wait(r, pb):
        pltpu.make_async_copy(acc.at[pl.ds(r * BLOCK, BLOCK)],
                              shared.at[pl.ds(pb * (NS * BLOCK) + s * BLOCK, BLOCK)],
                              sp).wait()

    def reduce_round(r, pb):
        moff = pb * (NS * BLOCK) + s * MSLICE
        for t in range(NS):
            pltpu.async_copy(shared.at[pl.ds(t * BLOCK + moff, MSLICE)],
                             rbuf.at[pl.ds(t * MSLICE, MSLICE)], sm)
        for t in range(NS):
            pltpu.make_async_copy(shared.at[pl.ds(t * BLOCK + moff, MSLICE)],
                                  rbuf.at[pl.ds(t * MSLICE, MSLICE)], sm).wait()

        @plsc.parallel_loop(0, MSLICE, step=LANES, unroll=2)
        def add_body(k0):
            v = rbuf[pl.ds(k0, LANES)]
            for t in range(1, NS):
                v += rbuf[pl.ds(t * MSLICE + k0, LANES)]
            rbuf[pl.ds(k0, LANES)] = v

        pltpu.sync_copy(
            rbuf.at[pl.ds(0, MSLICE)],
            out_hbm.at[pl.ds(c * NA_PAD + r * BLOCK + s * MSLICE, MSLICE)])

    pub(0, 0)

    def merge_pair(m, carry):
        for half in (0, 1):
            r = 2 * m + half
            pb = half
            pub_wait(r, pb)
            plsc.subcore_barrier()
            pub(r + 1, 1 - pb)
            reduce_round(r, pb)
            plsc.subcore_barrier()
        return carry

    lax.fori_loop(0, ROUNDS // 2, merge_pair, 0)

    if ROUNDS % 2 == 1:
        r_last = ROUNDS - 1
        pub_wait(r_last, 0)
        plsc.subcore_barrier()
        reduce_round(r_last, 0)


@functools.partial(
    pl.kernel,
    out_type=jax.ShapeDtypeStruct((NC * NA_PAD,), jnp.float32),
    mesh=plsc.VectorSubcoreMesh(core_axis_name="c", subcore_axis_name="s"),
    compiler_params=pltpu.CompilerParams(needs_layout_passes=False),
    scratch_types=[
        pltpu.VMEM((3, CHUNK), jnp.float32),
        pltpu.VMEM((3, CHUNK), jnp.float32),
        pltpu.VMEM((CHUNK,), jnp.int32),
        pltpu.VMEM((CHUNK,), jnp.int32),
        pltpu.VMEM((CHUNK,), jnp.int32),
        pltpu.VMEM((CHUNK,), jnp.int32),
        pltpu.VMEM((NA_PAD,), jnp.float32),
        pltpu.VMEM((NS * MSLICE,), jnp.float32),
        pltpu.VMEM_SHARED((2 * NS * BLOCK,), jnp.float32),
        pltpu.SemaphoreType.DMA,
        pltpu.SemaphoreType.DMA,
        pltpu.SemaphoreType.DMA,
        pltpu.SemaphoreType.DMA,
        pltpu.SemaphoreType.DMA,
        pltpu.SemaphoreType.DMA,
        pltpu.SemaphoreType.DMA,
        pltpu.SemaphoreType.DMA,
    ],
)
def _lj_kernel(dist_hbm, i_hbm, j_hbm, out_hbm, *scratch):
    _lj_body(dist_hbm, i_hbm, j_hbm, out_hbm, *scratch)


def kernel(distances, all_i, all_j):
    dist_t = distances.T
    partials = _lj_kernel(dist_t, all_i, all_j)
    partials = partials.reshape(NC, NA_PAD)
    energy = partials[0, :N_ATOMS] + partials[1, :N_ATOMS]
    return energy.reshape(-1, 1)
